# baseline (speedup 1.0000x reference)
import functools

import jax
import jax.numpy as jnp
from jax import lax
from jax.experimental import pallas as pl
from jax.experimental.pallas import tpu as pltpu
from jax.experimental.pallas import tpu_sc as plsc

B = 4096
H = 50
N = B * H
D = 64
NC = 2
NS = 16
NW = NC * NS
CH = 128
PER_W = N // NW
NCHUNK = PER_W // CH
L = 16
G = CH // L
V_CLS, V_SUB, V_WT = 30, 300, 5
SH_CLS, SH_SUB, SH_WT = 13, 18, 27
MASK_LOC, MASK_CLS, MASK_SUB, MASK_WT = 0x1FFF, 0x1F, 0x1FF, 0x1F


@functools.partial(
    pl.kernel,
    out_type=tuple(jax.ShapeDtypeStruct((N, D), jnp.float32) for _ in range(4)),
    mesh=plsc.VectorSubcoreMesh(core_axis_name="c", subcore_axis_name="s"),
    compiler_params=pltpu.CompilerParams(use_tc_tiling_on_sc=False,
                                         needs_layout_passes=False),
    scratch_types=[
        pltpu.VMEM((NCHUNK, CH), jnp.int32),
        pltpu.VMEM((2, CH), jnp.int32),
        pltpu.VMEM((2, 4, CH), jnp.int32),
        pltpu.VMEM((2, 4, CH, D), jnp.float32),
        pltpu.VMEM((V_CLS, D), jnp.float32),
        pltpu.VMEM((V_SUB, D), jnp.float32),
        pltpu.VMEM((V_WT, D), jnp.float32),
        pltpu.SemaphoreType.DMA((2,)),
        pltpu.SemaphoreType.DMA((2,)),
        pltpu.SemaphoreType.DMA,
    ],
)
def _sc_lookup(jobs, mpacked, tloc, tcls, tsub, twt,
               o0, o1, o2, o3, idx_v, pk_v, fid_v, rows_v,
               cls_v, sub_v, wt_v, sem_m, sem_l, sem_s):
    wid = lax.axis_index("s") * NC + lax.axis_index("c")
    base = wid * PER_W
    outs = (o0, o1, o2, o3)
    iota = lax.iota(jnp.int32, L)

    def meta_copy(k):
        buf = lax.rem(k, 2)
        return pltpu.make_async_copy(mpacked.at[idx_v.at[k]],
                                     pk_v.at[buf], sem_m.at[buf])

    def unpack_fids(k):
        buf = lax.rem(k, 2)
        for g in range(G):
            sl = pl.ds(g * L, L)
            p16 = pk_v[buf, sl]
            fid_v[buf, 0, sl] = p16 & MASK_LOC
            fid_v[buf, 1, sl] = lax.shift_right_logical(p16, SH_CLS) & MASK_CLS
            fid_v[buf, 2, sl] = lax.shift_right_logical(p16, SH_SUB) & MASK_SUB
            fid_v[buf, 3, sl] = lax.shift_right_logical(p16, SH_WT) & MASK_WT

    def loc_copy(k):
        buf = lax.rem(k, 2)
        return pltpu.make_async_copy(tloc.at[fid_v.at[buf, 0]],
                                     rows_v.at[buf, 0], sem_l.at[buf])

    def small_gathers(k):
        buf = lax.rem(k, 2)
        for f, tbl in ((1, cls_v), (2, sub_v), (3, wt_v)):
            dst = rows_v.at[buf, f]

            @plsc.parallel_loop(0, G * D, unroll=8)
            def gc_body(i):
                g = lax.shift_right_logical(i, 6)
                c = lax.bitwise_and(i, D - 1)
                vfid = fid_v[buf, f, pl.ds(g * L, L)]
                colc = jnp.full((L,), 0, jnp.int32) + c
                v = plsc.load_gather(tbl, [vfid, colc])
                plsc.store_scatter(dst, [iota + g * L, colc], v)

    def store_copies(k):
        buf = lax.rem(k, 2)
        return [pltpu.make_async_copy(rows_v.at[buf, f],
                                      outs[f].at[pl.ds(base + k * CH, CH)],
                                      sem_s)
                for f in range(4)]

    def fire(copies):
        for c in copies:
            c.start()

    def drain(copies):
        for c in copies:
            c.wait()

    pltpu.sync_copy(jobs.at[wid], idx_v)
    pltpu.sync_copy(tcls, cls_v)
    pltpu.sync_copy(tsub, sub_v)
    pltpu.sync_copy(twt, wt_v)

    meta_copy(0).start()
    meta_copy(1).start()
    meta_copy(0).wait()
    unpack_fids(0)
    loc_copy(0).start()
    small_gathers(0)

    def chunk_body(k, carry):

        @pl.when(k + 2 < NCHUNK)
        def _():
            meta_copy(k + 2).start()

        @pl.when(k >= 1)
        def _():
            with jax.named_scope("p_storewait"):
                drain(store_copies(k - 1))

        @pl.when(k + 1 < NCHUNK)
        def _():
            with jax.named_scope("p_metawait"):
                meta_copy(k + 1).wait()
            with jax.named_scope("p_unpack"):
                unpack_fids(k + 1)
            loc_copy(k + 1).start()
            with jax.named_scope("p_small"):
                small_gathers(k + 1)

        with jax.named_scope("p_locwait"):
            loc_copy(k).wait()
        fire(store_copies(k))
        return carry

    lax.fori_loop(0, NCHUNK, chunk_body, 0)
    drain(store_copies(NCHUNK - 1))


def kernel(job_ids, metadata_table, loc_emb, cls_emb, sub_emb, wt_emb):
    jobs = job_ids.reshape(NW, NCHUNK, CH).astype(jnp.int32)
    mpacked = (metadata_table[:, 0]
               + (metadata_table[:, 1] << SH_CLS)
               + (metadata_table[:, 2] << SH_SUB)
               + (metadata_table[:, 3] << SH_WT))
    outs = _sc_lookup(jobs, mpacked, loc_emb, cls_emb, sub_emb, wt_emb)
    return tuple(o.reshape(B, H, D) for o in outs)


# speedup vs baseline: 1.8455x; 1.8455x over previous
import functools

import jax
import jax.numpy as jnp
from jax import lax
from jax.experimental import pallas as pl
from jax.experimental.pallas import tpu as pltpu
from jax.experimental.pallas import tpu_sc as plsc

B = 4096
H = 50
N = B * H
D = 64
NC = 2
NS = 16
NW = NC * NS
CH = 128
PER_W = N // NW
NCHUNK = PER_W // CH
L = 16
G = CH // L
V_CLS, V_SUB, V_WT = 30, 300, 5
SH_CLS, SH_SUB, SH_WT = 13, 18, 27
MASK_LOC, MASK_CLS, MASK_SUB, MASK_WT = 0x1FFF, 0x1F, 0x1FF, 0x1F


@functools.partial(
    pl.kernel,
    out_type=tuple(jax.ShapeDtypeStruct((N, D), jnp.float32) for _ in range(4)),
    mesh=plsc.VectorSubcoreMesh(core_axis_name="c", subcore_axis_name="s"),
    compiler_params=pltpu.CompilerParams(use_tc_tiling_on_sc=False,
                                         needs_layout_passes=False),
    scratch_types=[
        pltpu.VMEM((NCHUNK, CH), jnp.int32),
        pltpu.VMEM((2, CH), jnp.int32),
        pltpu.VMEM((2, 4, CH), jnp.int32),
        pltpu.VMEM((2, CH, D), jnp.float32),
        pltpu.VMEM((2, 3, CH, D + 1), jnp.float32),
        pltpu.VMEM((V_CLS, D + 1), jnp.float32),
        pltpu.VMEM((V_SUB, D + 1), jnp.float32),
        pltpu.VMEM((V_WT, D + 1), jnp.float32),
        pltpu.SemaphoreType.DMA((2,)),
        pltpu.SemaphoreType.DMA((2,)),
        pltpu.SemaphoreType.DMA,
    ],
)
def _sc_lookup(jobs, mpacked, tloc, tcls, tsub, twt,
               o0, o1, o2, o3, idx_v, pk_v, fid_v, loc_v, small_v,
               cls_v, sub_v, wt_v, sem_m, sem_l, sem_s):
    wid = lax.axis_index("s") * NC + lax.axis_index("c")
    base = wid * PER_W
    outs = (o0, o1, o2, o3)
    iota = lax.iota(jnp.int32, L)

    def meta_copy(k):
        buf = lax.rem(k, 2)
        return pltpu.make_async_copy(mpacked.at[idx_v.at[k]],
                                     pk_v.at[buf], sem_m.at[buf])

    def unpack_fids(k):
        buf = lax.rem(k, 2)
        for g in range(G):
            sl = pl.ds(g * L, L)
            p16 = pk_v[buf, sl]
            fid_v[buf, 0, sl] = p16 & MASK_LOC
            fid_v[buf, 1, sl] = lax.shift_right_logical(p16, SH_CLS) & MASK_CLS
            fid_v[buf, 2, sl] = lax.shift_right_logical(p16, SH_SUB) & MASK_SUB
            fid_v[buf, 3, sl] = lax.shift_right_logical(p16, SH_WT) & MASK_WT

    def loc_copy(k):
        buf = lax.rem(k, 2)
        return pltpu.make_async_copy(tloc.at[fid_v.at[buf, 0]],
                                     loc_v.at[buf], sem_l.at[buf])

    def small_gathers(k):
        buf = lax.rem(k, 2)
        for f, tbl in ((1, cls_v), (2, sub_v), (3, wt_v)):
            dst = small_v.at[buf, f - 1]

            @plsc.parallel_loop(0, G * D, unroll=8)
            def gc_body(i):
                g = lax.shift_right_logical(i, 6)
                c = lax.bitwise_and(i, D - 1)
                vfid = fid_v[buf, f, pl.ds(g * L, L)]
                colc = jnp.full((L,), 0, jnp.int32) + c
                v = plsc.load_gather(tbl, [vfid, colc])
                plsc.store_scatter(dst, [iota + g * L, colc], v)

    def store_copies(k):
        buf = lax.rem(k, 2)
        cps = [pltpu.make_async_copy(loc_v.at[buf],
                                     outs[0].at[pl.ds(base + k * CH, CH)],
                                     sem_s)]
        for f in range(1, 4):
            cps.append(pltpu.make_async_copy(
                small_v.at[buf, f - 1, :, pl.ds(0, D)],
                outs[f].at[pl.ds(base + k * CH, CH)], sem_s))
        return cps

    def fire(copies):
        for c in copies:
            c.start()

    def drain(copies):
        for c in copies:
            c.wait()

    pltpu.sync_copy(jobs.at[wid], idx_v)
    pltpu.sync_copy(tcls, cls_v)
    pltpu.sync_copy(tsub, sub_v)
    pltpu.sync_copy(twt, wt_v)

    meta_copy(0).start()
    meta_copy(1).start()
    meta_copy(0).wait()
    unpack_fids(0)
    loc_copy(0).start()
    small_gathers(0)

    def chunk_body(k, carry):

        @pl.when(k + 2 < NCHUNK)
        def _():
            meta_copy(k + 2).start()

        @pl.when(k >= 1)
        def _():
            with jax.named_scope("p_storewait"):
                drain(store_copies(k - 1))

        @pl.when(k + 1 < NCHUNK)
        def _():
            with jax.named_scope("p_metawait"):
                meta_copy(k + 1).wait()
            with jax.named_scope("p_unpack"):
                unpack_fids(k + 1)
            loc_copy(k + 1).start()
            with jax.named_scope("p_small"):
                small_gathers(k + 1)

        with jax.named_scope("p_locwait"):
            loc_copy(k).wait()
        fire(store_copies(k))
        return carry

    lax.fori_loop(0, NCHUNK, chunk_body, 0)
    drain(store_copies(NCHUNK - 1))


def kernel(job_ids, metadata_table, loc_emb, cls_emb, sub_emb, wt_emb):
    jobs = job_ids.reshape(NW, NCHUNK, CH).astype(jnp.int32)
    mpacked = (metadata_table[:, 0]
               + (metadata_table[:, 1] << SH_CLS)
               + (metadata_table[:, 2] << SH_SUB)
               + (metadata_table[:, 3] << SH_WT))
    pad1 = ((0, 0), (0, 1))
    outs = _sc_lookup(jobs, mpacked, loc_emb, jnp.pad(cls_emb, pad1),
                      jnp.pad(sub_emb, pad1), jnp.pad(wt_emb, pad1))
    return tuple(o.reshape(B, H, D) for o in outs)


# speedup vs baseline: 2.4494x; 1.3272x over previous
import functools

import jax
import jax.numpy as jnp
from jax import lax
from jax.experimental import pallas as pl
from jax.experimental.pallas import tpu as pltpu
from jax.experimental.pallas import tpu_sc as plsc

B = 4096
H = 50
N = B * H
D = 64
NC = 2
NS = 16
NW = NC * NS
CH = 128
PER_W = N // NW
NCHUNK = PER_W // CH
L = 16
G = CH // L
V_CLS, V_SUB, V_WT = 30, 300, 5
SH_CLS, SH_SUB, SH_WT = 13, 18, 27
MASK_LOC, MASK_CLS, MASK_SUB, MASK_WT = 0x1FFF, 0x1F, 0x1FF, 0x1F


@functools.partial(
    pl.kernel,
    out_type=(jax.ShapeDtypeStruct((H, B, D), jnp.float32),)
             + tuple(jax.ShapeDtypeStruct((H, D, B), jnp.float32)
                     for _ in range(3)),
    mesh=plsc.VectorSubcoreMesh(core_axis_name="c", subcore_axis_name="s"),
    compiler_params=pltpu.CompilerParams(use_tc_tiling_on_sc=False,
                                         needs_layout_passes=False),
    scratch_types=[
        pltpu.VMEM((NCHUNK, CH), jnp.int32),
        pltpu.VMEM((2, CH), jnp.int32),
        pltpu.VMEM((2, 4, CH), jnp.int32),
        pltpu.VMEM((2, CH, D), jnp.float32),
        pltpu.VMEM((2, 3, D, CH), jnp.float32),
        pltpu.VMEM((V_CLS, D + 1), jnp.float32),
        pltpu.VMEM((V_SUB, D + 1), jnp.float32),
        pltpu.VMEM((V_WT, D + 1), jnp.float32),
        pltpu.SemaphoreType.DMA((2,)),
        pltpu.SemaphoreType.DMA((2,)),
        pltpu.SemaphoreType.DMA,
    ],
)
def _sc_lookup(jobs, mpacked, tloc, tcls, tsub, twt,
               o0, o1, o2, o3, idx_v, pk_v, fid_v, loc_v, small_v,
               cls_v, sub_v, wt_v, sem_m, sem_l, sem_s):
    wid = lax.axis_index("s") * NC + lax.axis_index("c")
    bbase = wid * CH
    outs = (o0, o1, o2, o3)
    iota = lax.iota(jnp.int32, L)

    def meta_copy(k):
        buf = lax.rem(k, 2)
        return pltpu.make_async_copy(mpacked.at[idx_v.at[k]],
                                     pk_v.at[buf], sem_m.at[buf])

    def unpack_fids(k):
        buf = lax.rem(k, 2)
        for g in range(G):
            sl = pl.ds(g * L, L)
            p16 = pk_v[buf, sl]
            fid_v[buf, 0, sl] = p16 & MASK_LOC
            fid_v[buf, 1, sl] = lax.shift_right_logical(p16, SH_CLS) & MASK_CLS
            fid_v[buf, 2, sl] = lax.shift_right_logical(p16, SH_SUB) & MASK_SUB
            fid_v[buf, 3, sl] = lax.shift_right_logical(p16, SH_WT) & MASK_WT

    def loc_copy(k):
        buf = lax.rem(k, 2)
        return pltpu.make_async_copy(tloc.at[fid_v.at[buf, 0]],
                                     loc_v.at[buf], sem_l.at[buf])

    def small_gathers(k):
        buf = lax.rem(k, 2)
        for f, tbl in ((1, cls_v), (2, sub_v), (3, wt_v)):
            dst = small_v.at[buf, f - 1]

            @plsc.parallel_loop(0, G * D, unroll=8)
            def gc_body(i):
                g = lax.shift_right_logical(i, 6)
                c = lax.bitwise_and(i, D - 1)
                vfid = fid_v[buf, f, pl.ds(g * L, L)]
                colc = jnp.full((L,), 0, jnp.int32) + c
                v = plsc.load_gather(tbl, [vfid, colc])
                dst[c, pl.ds(g * L, L)] = v

    def store_copies(k):
        buf = lax.rem(k, 2)
        cps = [pltpu.make_async_copy(loc_v.at[buf],
                                     outs[0].at[k, pl.ds(bbase, CH)],
                                     sem_s)]
        for f in range(1, 4):
            cps.append(pltpu.make_async_copy(
                small_v.at[buf, f - 1],
                outs[f].at[k, slice(None), pl.ds(bbase, CH)], sem_s))
        return cps

    def fire(copies):
        for c in copies:
            c.start()

    def drain(copies):
        for c in copies:
            c.wait()

    pltpu.sync_copy(jobs.at[wid], idx_v)
    pltpu.sync_copy(tcls, cls_v)
    pltpu.sync_copy(tsub, sub_v)
    pltpu.sync_copy(twt, wt_v)

    meta_copy(0).start()
    meta_copy(1).start()
    meta_copy(0).wait()
    unpack_fids(0)
    loc_copy(0).start()
    small_gathers(0)

    def chunk_body(k, carry):

        @pl.when(k + 2 < NCHUNK)
        def _():
            meta_copy(k + 2).start()

        @pl.when(k >= 1)
        def _():
            with jax.named_scope("p_storewait"):
                drain(store_copies(k - 1))

        @pl.when(k + 1 < NCHUNK)
        def _():
            with jax.named_scope("p_metawait"):
                meta_copy(k + 1).wait()
            with jax.named_scope("p_unpack"):
                unpack_fids(k + 1)
            loc_copy(k + 1).start()
            with jax.named_scope("p_small"):
                small_gathers(k + 1)

        with jax.named_scope("p_locwait"):
            loc_copy(k).wait()
        fire(store_copies(k))
        return carry

    lax.fori_loop(0, NCHUNK, chunk_body, 0)
    drain(store_copies(NCHUNK - 1))


def kernel(job_ids, metadata_table, loc_emb, cls_emb, sub_emb, wt_emb):
    jobs = job_ids.T.reshape(NCHUNK, NW, CH).transpose(1, 0, 2).astype(jnp.int32)
    mpacked = (metadata_table[:, 0]
               + (metadata_table[:, 1] << SH_CLS)
               + (metadata_table[:, 2] << SH_SUB)
               + (metadata_table[:, 3] << SH_WT))
    pad1 = ((0, 0), (0, 1))
    outs = _sc_lookup(jobs, mpacked, loc_emb, jnp.pad(cls_emb, pad1),
                      jnp.pad(sub_emb, pad1), jnp.pad(wt_emb, pad1))
    return (outs[0].transpose(1, 0, 2),) + tuple(
        o.transpose(2, 0, 1) for o in outs[1:])
